# baseline (speedup 1.0000x reference)
import functools

import jax
import jax.numpy as jnp
from jax import lax
from jax.experimental import pallas as pl
from jax.experimental.pallas import tpu as pltpu
from jax.experimental.pallas import tpu_sc as plsc

N = 10000
G = 16
NC, NS = 2, 16
NW = NC * NS
C = 128
NP = 10240
RPS = NP // NS

_mesh = plsc.VectorSubcoreMesh(core_axis_name="c", subcore_axis_name="s")


def _make_fused1_kernel(chunks, F):
    @functools.partial(
        pl.kernel,
        out_type=[jax.ShapeDtypeStruct((NC, NP, F), jnp.float32),
                  jax.ShapeDtypeStruct((NP,), jnp.float32)],
        mesh=_mesh,
        compiler_params=pltpu.CompilerParams(needs_layout_passes=False, use_tc_tiling_on_sc=False, skip_device_barrier=True, disable_bounds_checks=True),
        scratch_types=[
            pltpu.VMEM((chunks, C), jnp.int32),
            pltpu.VMEM((2, chunks, C), jnp.int32),
            pltpu.VMEM((2, chunks, C), jnp.float32),
            pltpu.VMEM((C,), jnp.float32),
            pltpu.VMEM((NP,), jnp.float32),
            pltpu.VMEM((C, F), jnp.float32),
            pltpu.VMEM((C, F), jnp.float32),
            pltpu.VMEM((C, F), jnp.float32),
            pltpu.VMEM((C, F), jnp.float32),
            pltpu.SemaphoreType.DMA,
            pltpu.SemaphoreType.DMA,
            pltpu.SemaphoreType.DMA,
            pltpu.SemaphoreType.DMA,
            pltpu.SemaphoreType.DMA,
            pltpu.SemaphoreType.DMA,
            pltpu.SemaphoreType.DMA,
            pltpu.SemaphoreType.DMA,
            pltpu.VMEM_SHARED((NP,), jnp.float32),
            pltpu.VMEM_SHARED((NP, F), jnp.float32),
        ],
    )
    def fused1(xw_hbm, src_hbm, dst_hbm, w_hbm, zeros1_hbm, zerosf_hbm,
               out_hbm, dinv_hbm, src_v, dst2_v, w2_v, swb_v, dinv_t,
               r0, r1, r2, r3, g0, g1, g2, g3, s0, s1, s2, s3, dsh, acc):
        c = lax.axis_index("c")
        s = lax.axis_index("s")
        sl = pl.ds(s * RPS, RPS)
        pltpu.sync_copy(src_hbm.at[2 * s + c], src_v)
        pltpu.sync_copy(dst_hbm.at[2 * s], dst2_v.at[0])
        pltpu.sync_copy(dst_hbm.at[2 * s + 1], dst2_v.at[1])
        pltpu.sync_copy(w_hbm.at[2 * s], w2_v.at[0])
        pltpu.sync_copy(w_hbm.at[2 * s + 1], w2_v.at[1])
        pltpu.sync_copy(zeros1_hbm.at[sl], dsh.at[sl])
        pltpu.sync_copy(zerosf_hbm.at[sl], acc.at[sl])
        plsc.subcore_barrier()

        zl = jnp.zeros((16,), jnp.int32)
        rows = (r0, r1, r2, r3)
        gsem = (g0, g1, g2, g3)
        ssem = (s0, s1, s2, s3)

        def gather(j, b):
            pltpu.async_copy(xw_hbm.at[src_v.at[j]], rows[b], gsem[b])

        gather(0, 0)
        gather(1, 1)

        def deground(r, carry):
            cps = [pltpu.async_copy(w2_v.at[q, 2 * r + b],
                                    dsh.at[dst2_v.at[q, 2 * r + b]],
                                    ssem[2 * q + b], add=True)
                   for q in range(2) for b in range(2)]
            for cp in cps:
                cp.wait()
            return carry

        lax.fori_loop(0, chunks // 2, deground, 0)
        plsc.subcore_barrier()

        pltpu.sync_copy(dsh.at[sl], dinv_t.at[sl])

        def newton(k, carry):
            off = pl.ds(s * RPS + k * 16, 16)
            d = dinv_t[off] + 1.0
            i = plsc.bitcast(d, jnp.int32)
            r_ = plsc.bitcast(jnp.int32(0x5F3759DF) - (i >> 1), jnp.float32)
            for _ in range(3):
                r_ = r_ * (1.5 - 0.5 * d * r_ * r_)
            dinv_t[off] = jnp.where(d > 0, r_, 0.0)
            return carry

        lax.fori_loop(0, RPS // 16, newton, 0, unroll=4)
        pltpu.sync_copy(dinv_t.at[sl], dsh.at[sl])

        @pl.when(c == 0)
        def _():
            pltpu.sync_copy(dinv_t.at[sl], dinv_hbm.at[sl])

        plsc.subcore_barrier()
        pltpu.sync_copy(dsh, dinv_t)

        def gwait(j, b):
            pltpu.make_async_copy(xw_hbm.at[src_v.at[j]],
                                  rows[b], gsem[b]).wait()

        def scatter(j, b):
            pltpu.async_copy(rows[b], acc.at[dst2_v.at[c, j]],
                             ssem[b], add=True)

        def swait(j, b):
            pltpu.make_async_copy(rows[b], acc.at[dst2_v.at[c, j]],
                                  ssem[b]).wait()

        def scale(j, b):
            for k in range(C // 16):
                kk = pl.ds(16 * k, 16)
                sv = src_v[j, kk]
                dv = plsc.load_gather(dinv_t, [sv])
                swb_v[kk] = w2_v[c, j, kk] * dv

            def body(g, cc):
                e0 = g * 8
                sws = [plsc.load_gather(swb_v, [zl + (e0 + i)])
                       for i in range(8)]
                for i in range(8):
                    for f0 in range(0, F, 16):
                        rows[b][e0 + i, pl.ds(f0, 16)] = (
                            rows[b][e0 + i, pl.ds(f0, 16)] * sws[i])
                return cc

            lax.fori_loop(0, C // 8, body, 0)

        for j in range(4):
            gwait(j, j)
            scale(j, j)
            scatter(j, j)
            if j < 2:
                gather(j + 2, j + 2)
            else:
                swait(j - 2, j - 2)
                gather(j + 2, j - 2)

        def round_(r, carry):
            for b in range(4):
                j = r * 4 + b
                gwait(j, b)
                scale(j, b)
                scatter(j, b)
                bn = (b + 2) % 4

                @pl.when(j + 2 < chunks)
                def _():
                    swait(j - 2, bn)
                    gather(j + 2, bn)
            return carry

        lax.fori_loop(1, chunks // 4, round_, 0)
        for b in range(4):
            swait(chunks - 4 + b, b)
        plsc.subcore_barrier()
        pltpu.sync_copy(acc.at[sl], out_hbm.at[c, sl])

    return fused1


def _make_mp_kernel(chunks, F):
    @functools.partial(
        pl.kernel,
        out_type=jax.ShapeDtypeStruct((NC, NP, F), jnp.float32),
        mesh=_mesh,
        compiler_params=pltpu.CompilerParams(needs_layout_passes=False, use_tc_tiling_on_sc=False, skip_device_barrier=True, disable_bounds_checks=True),
        scratch_types=[
            pltpu.VMEM((chunks, C), jnp.int32),
            pltpu.VMEM((chunks, C), jnp.int32),
            pltpu.VMEM((chunks, C), jnp.float32),
            pltpu.VMEM((C, F), jnp.float32),
            pltpu.VMEM((C, F), jnp.float32),
            pltpu.VMEM((C, F), jnp.float32),
            pltpu.VMEM((C, F), jnp.float32),
            pltpu.SemaphoreType.DMA,
            pltpu.SemaphoreType.DMA,
            pltpu.SemaphoreType.DMA,
            pltpu.SemaphoreType.DMA,
            pltpu.SemaphoreType.DMA,
            pltpu.SemaphoreType.DMA,
            pltpu.SemaphoreType.DMA,
            pltpu.SemaphoreType.DMA,
            pltpu.VMEM_SHARED((NP, F), jnp.float32),
        ],
    )
    def mp_kernel(y_hbm, src_hbm, dst_hbm, w_hbm, zeros_hbm, out_hbm,
                  src_v, dst_v, w_v, r0, r1, r2, r3,
                  g0, g1, g2, g3, s0, s1, s2, s3, acc):
        c = lax.axis_index("c")
        s = lax.axis_index("s")
        wid = c * NS + s
        pltpu.sync_copy(src_hbm.at[wid], src_v)
        pltpu.sync_copy(dst_hbm.at[wid], dst_v)
        pltpu.sync_copy(w_hbm.at[wid], w_v)
        pltpu.sync_copy(zeros_hbm.at[pl.ds(s * RPS, RPS)],
                        acc.at[pl.ds(s * RPS, RPS)])
        plsc.subcore_barrier()

        zl = jnp.zeros((16,), jnp.int32)
        rows = (r0, r1, r2, r3)
        gsem = (g0, g1, g2, g3)
        ssem = (s0, s1, s2, s3)

        def gather(j, b):
            pltpu.async_copy(y_hbm.at[src_v.at[j]], rows[b], gsem[b])

        def gwait(j, b):
            pltpu.make_async_copy(y_hbm.at[src_v.at[j]],
                                  rows[b], gsem[b]).wait()

        def scatter(j, b):
            pltpu.async_copy(rows[b], acc.at[dst_v.at[j]], ssem[b], add=True)

        def swait(j, b):
            pltpu.make_async_copy(rows[b], acc.at[dst_v.at[j]],
                                  ssem[b]).wait()

        def scale(j, b):
            jvec = zl + j

            def body(g, cc):
                e0 = g * 8
                sws = [plsc.load_gather(w_v, [jvec, zl + (e0 + i)])
                       for i in range(8)]
                for i in range(8):
                    for f0 in range(0, F, 16):
                        rows[b][e0 + i, pl.ds(f0, 16)] = (
                            rows[b][e0 + i, pl.ds(f0, 16)] * sws[i])
                return cc

            lax.fori_loop(0, C // 8, body, 0)

        gather(0, 0)
        gather(1, 1)
        for j in range(4):
            gwait(j, j)
            scale(j, j)
            scatter(j, j)
            if j < 2:
                gather(j + 2, j + 2)
            else:
                swait(j - 2, j - 2)
                gather(j + 2, j - 2)

        def round_(r, carry):
            for b in range(4):
                j = r * 4 + b
                gwait(j, b)
                scale(j, b)
                scatter(j, b)

                bn = (b + 2) % 4

                @pl.when(j + 2 < chunks)
                def _():
                    swait(j - 2, bn)
                    gather(j + 2, bn)
            return carry

        lax.fori_loop(1, chunks // 4, round_, 0)
        for b in range(4):
            swait(chunks - 4 + b, b)
        plsc.subcore_barrier()
        pltpu.sync_copy(acc.at[pl.ds(s * RPS, RPS)],
                        out_hbm.at[c, pl.ds(s * RPS, RPS)])

    return mp_kernel


_HI = lax.Precision.HIGHEST


def _dot(a, b):
    return lax.dot_general(a, b, (((1,), (0,)), ((), ())),
                           preferred_element_type=jnp.float32)


def _bn_relu(h):
    m = jnp.mean(h, axis=0, keepdims=True)
    cd = h - m
    v = jnp.mean(cd * cd, axis=0, keepdims=True)
    return jnp.maximum(cd / jnp.sqrt(v + 1e-5), 0.0)


def _tca_body(x_ref, w1_ref, xw_ref):
    xw_ref[...] = _dot(x_ref[...], w1_ref[...])


def _tc2_body(sp_ref, xw1_ref, dinv_ref, b1_ref, w2_ref, xw2_ref, y2_ref):
    dinv = dinv_ref[...]
    out = dinv * (sp_ref[0] + sp_ref[1]) + dinv * dinv * xw1_ref[...] \
        + b1_ref[...]
    h = _bn_relu(out)
    xw2 = _dot(h, w2_ref[...])
    xw2_ref[...] = xw2
    y2_ref[...] = xw2 * dinv


def _tc3_body(sp_ref, xw2_ref, dinv_ref, b2_ref, batch_ref,
              fw1a_ref, fw1b_ref, fb1_ref, fw2_ref, fb2_ref, z_ref):
    dinv = dinv_ref[...]
    out = dinv * (sp_ref[0] + sp_ref[1]) + dinv * dinv * xw2_ref[...] \
        + b2_ref[...]
    h = _bn_relu(out)
    b = batch_ref[...]
    iot = lax.broadcasted_iota(jnp.int32, (N, G), 1)
    oh = (b == iot).astype(jnp.float32)
    sums = lax.dot_general(oh, h, (((0,), (0,)), ((), ())),
                           precision=_HI, preferred_element_type=jnp.float32)
    ones = jnp.ones((N, 1), jnp.float32)
    cnt = lax.dot_general(oh, ones, (((0,), (0,)), ((), ())),
                          precision=_HI, preferred_element_type=jnp.float32)
    mean = sums / jnp.maximum(cnt, 1.0)
    neg = jnp.float32(float("-inf"))
    mxs = [jnp.max(jnp.where(b == g, h, neg), axis=0, keepdims=True)
           for g in range(G)]
    mx = jnp.concatenate(mxs, axis=0)
    z1 = _dot(mean, fw1a_ref[...]) + _dot(mx, fw1b_ref[...]) + fb1_ref[...]
    m2 = jnp.mean(z1, axis=0, keepdims=True)
    c2 = z1 - m2
    v2 = jnp.mean(c2 * c2, axis=0, keepdims=True)
    z2 = jnp.maximum(c2 / jnp.sqrt(v2 + 1e-5), 0.0)
    z_ref[...] = _dot(z2, fw2_ref[...]) + fb2_ref[...]


def _sd(shape):
    return jax.ShapeDtypeStruct(shape, jnp.float32)


def kernel(x, edge_index, edge_weight, batch, W1, b1, W2, b2,
           fW1, fb1, fW2, fb2):
    E = edge_weight.shape[0]
    H = W1.shape[1]
    chunks = -(-E // (NW * C))
    chunks += chunks % 2
    epad = NW * chunks * C
    pad = epad - E

    src = edge_index[0].astype(jnp.int32)
    dst = edge_index[1].astype(jnp.int32)
    w = edge_weight.astype(jnp.float32)
    ar = jnp.arange(pad, dtype=jnp.int32)
    src_p = jnp.concatenate([src, ar % N]).reshape(NW, chunks, C)
    dst_p = jnp.concatenate([dst, N + ar % (NP - N)]).reshape(NW, chunks, C)
    w_p = jnp.concatenate([w, jnp.zeros((pad,), jnp.float32)]
                          ).reshape(NW, chunks, C)

    zeros1 = jnp.zeros((NP,), jnp.float32)
    zeros_h = jnp.zeros((NP, H), jnp.float32)
    zeros_2h = jnp.zeros((NP, 2 * H), jnp.float32)

    xw1 = pl.pallas_call(_tca_body, out_shape=_sd((N, H)))(x, W1)
    s1, dinv_np = _make_fused1_kernel(chunks, H)(
        xw1, src_p, dst_p, w_p, zeros1, zeros_h)
    dinv = dinv_np[:N].reshape(N, 1)
    xw2, y2 = pl.pallas_call(
        _tc2_body,
        out_shape=[_sd((N, 2 * H)), _sd((N, 2 * H))],
    )(s1[:, :N, :], xw1, dinv, b1.reshape(1, H), W2)

    s2 = _make_mp_kernel(chunks, 2 * H)(y2, src_p, dst_p, w_p, zeros_2h)
    z = pl.pallas_call(
        _tc3_body,
        out_shape=_sd((G, 2)),
    )(s2[:, :N, :], xw2, dinv, b2.reshape(1, 2 * H),
      batch.reshape(N, 1).astype(jnp.int32),
      fW1[:2 * H], fW1[2 * H:], fb1.reshape(1, H), fW2, fb2.reshape(1, 2))
    return z


# speedup vs baseline: 1.0013x; 1.0013x over previous
import functools

import jax
import jax.numpy as jnp
from jax import lax
from jax.experimental import pallas as pl
from jax.experimental.pallas import tpu as pltpu
from jax.experimental.pallas import tpu_sc as plsc

N = 10000
G = 16
NC, NS = 2, 16
NW = NC * NS
C = 128
NP = 10240
RPS = NP // NS

_mesh = plsc.VectorSubcoreMesh(core_axis_name="c", subcore_axis_name="s")


def _make_fused1_kernel(chunks, F):
    @functools.partial(
        pl.kernel,
        out_type=[jax.ShapeDtypeStruct((NC, NP, F), jnp.float32),
                  jax.ShapeDtypeStruct((NP,), jnp.float32)],
        mesh=_mesh,
        compiler_params=pltpu.CompilerParams(needs_layout_passes=False, use_tc_tiling_on_sc=False),
        scratch_types=[
            pltpu.VMEM((chunks, C), jnp.int32),
            pltpu.VMEM((2, chunks, C), jnp.int32),
            pltpu.VMEM((2, chunks, C), jnp.float32),
            pltpu.VMEM((C,), jnp.float32),
            pltpu.VMEM((NP,), jnp.float32),
            pltpu.VMEM((C, F), jnp.float32),
            pltpu.VMEM((C, F), jnp.float32),
            pltpu.VMEM((C, F), jnp.float32),
            pltpu.VMEM((C, F), jnp.float32),
            pltpu.SemaphoreType.DMA,
            pltpu.SemaphoreType.DMA,
            pltpu.SemaphoreType.DMA,
            pltpu.SemaphoreType.DMA,
            pltpu.SemaphoreType.DMA,
            pltpu.SemaphoreType.DMA,
            pltpu.SemaphoreType.DMA,
            pltpu.SemaphoreType.DMA,
            pltpu.VMEM_SHARED((NP,), jnp.float32),
            pltpu.VMEM_SHARED((NP, F), jnp.float32),
        ],
    )
    def fused1(xw_hbm, src_hbm, dst_hbm, w_hbm, zeros1_hbm, zerosf_hbm,
               out_hbm, dinv_hbm, src_v, dst2_v, w2_v, swb_v, dinv_t,
               r0, r1, r2, r3, g0, g1, g2, g3, s0, s1, s2, s3, dsh, acc):
        c = lax.axis_index("c")
        s = lax.axis_index("s")
        sl = pl.ds(s * RPS, RPS)
        pltpu.sync_copy(src_hbm.at[2 * s + c], src_v)
        pltpu.sync_copy(dst_hbm.at[2 * s], dst2_v.at[0])
        pltpu.sync_copy(dst_hbm.at[2 * s + 1], dst2_v.at[1])
        pltpu.sync_copy(w_hbm.at[2 * s], w2_v.at[0])
        pltpu.sync_copy(w_hbm.at[2 * s + 1], w2_v.at[1])
        pltpu.sync_copy(zeros1_hbm.at[sl], dsh.at[sl])
        pltpu.sync_copy(zerosf_hbm.at[sl], acc.at[sl])
        plsc.subcore_barrier()

        zl = jnp.zeros((16,), jnp.int32)
        rows = (r0, r1, r2, r3)
        gsem = (g0, g1, g2, g3)
        ssem = (s0, s1, s2, s3)

        def gather(j, b):
            pltpu.async_copy(xw_hbm.at[src_v.at[j]], rows[b], gsem[b])

        gather(0, 0)
        gather(1, 1)

        def deground(r, carry):
            cps = [pltpu.async_copy(w2_v.at[q, 2 * r + b],
                                    dsh.at[dst2_v.at[q, 2 * r + b]],
                                    ssem[2 * q + b], add=True)
                   for q in range(2) for b in range(2)]
            for cp in cps:
                cp.wait()
            return carry

        lax.fori_loop(0, chunks // 2, deground, 0)
        plsc.subcore_barrier()

        pltpu.sync_copy(dsh.at[sl], dinv_t.at[sl])

        def newton(k, carry):
            off = pl.ds(s * RPS + k * 16, 16)
            d = dinv_t[off] + 1.0
            i = plsc.bitcast(d, jnp.int32)
            r_ = plsc.bitcast(jnp.int32(0x5F3759DF) - (i >> 1), jnp.float32)
            for _ in range(3):
                r_ = r_ * (1.5 - 0.5 * d * r_ * r_)
            dinv_t[off] = jnp.where(d > 0, r_, 0.0)
            return carry

        lax.fori_loop(0, RPS // 16, newton, 0, unroll=4)
        pltpu.sync_copy(dinv_t.at[sl], dsh.at[sl])

        @pl.when(c == 0)
        def _():
            pltpu.sync_copy(dinv_t.at[sl], dinv_hbm.at[sl])

        plsc.subcore_barrier()
        pltpu.sync_copy(dsh, dinv_t)

        def gwait(j, b):
            pltpu.make_async_copy(xw_hbm.at[src_v.at[j]],
                                  rows[b], gsem[b]).wait()

        def scatter(j, b):
            pltpu.async_copy(rows[b], acc.at[dst2_v.at[c, j]],
                             ssem[b], add=True)

        def swait(j, b):
            pltpu.make_async_copy(rows[b], acc.at[dst2_v.at[c, j]],
                                  ssem[b]).wait()

        def scale(j, b):
            for k in range(C // 16):
                kk = pl.ds(16 * k, 16)
                sv = src_v[j, kk]
                dv = plsc.load_gather(dinv_t, [sv])
                swb_v[kk] = w2_v[c, j, kk] * dv

            def body(g, cc):
                e0 = g * 8
                sws = [plsc.load_gather(swb_v, [zl + (e0 + i)])
                       for i in range(8)]
                for i in range(8):
                    for f0 in range(0, F, 16):
                        rows[b][e0 + i, pl.ds(f0, 16)] = (
                            rows[b][e0 + i, pl.ds(f0, 16)] * sws[i])
                return cc

            lax.fori_loop(0, C // 8, body, 0)

        for j in range(4):
            gwait(j, j)
            scale(j, j)
            scatter(j, j)
            if j < 2:
                gather(j + 2, j + 2)
            else:
                swait(j - 2, j - 2)
                gather(j + 2, j - 2)

        def round_(r, carry):
            for b in range(4):
                j = r * 4 + b
                gwait(j, b)
                scale(j, b)
                scatter(j, b)
                bn = (b + 2) % 4

                @pl.when(j + 2 < chunks)
                def _():
                    swait(j - 2, bn)
                    gather(j + 2, bn)
            return carry

        lax.fori_loop(1, chunks // 4, round_, 0)
        for b in range(4):
            swait(chunks - 4 + b, b)
        plsc.subcore_barrier()
        pltpu.sync_copy(acc.at[sl], out_hbm.at[c, sl])

    return fused1


def _make_mp_kernel(chunks, F):
    @functools.partial(
        pl.kernel,
        out_type=jax.ShapeDtypeStruct((NC, NP, F), jnp.float32),
        mesh=_mesh,
        compiler_params=pltpu.CompilerParams(needs_layout_passes=False, use_tc_tiling_on_sc=False),
        scratch_types=[
            pltpu.VMEM((chunks, C), jnp.int32),
            pltpu.VMEM((chunks, C), jnp.int32),
            pltpu.VMEM((chunks, C), jnp.float32),
            pltpu.VMEM((C, F), jnp.float32),
            pltpu.VMEM((C, F), jnp.float32),
            pltpu.VMEM((C, F), jnp.float32),
            pltpu.VMEM((C, F), jnp.float32),
            pltpu.SemaphoreType.DMA,
            pltpu.SemaphoreType.DMA,
            pltpu.SemaphoreType.DMA,
            pltpu.SemaphoreType.DMA,
            pltpu.SemaphoreType.DMA,
            pltpu.SemaphoreType.DMA,
            pltpu.SemaphoreType.DMA,
            pltpu.SemaphoreType.DMA,
            pltpu.VMEM_SHARED((NP, F), jnp.float32),
        ],
    )
    def mp_kernel(y_hbm, src_hbm, dst_hbm, w_hbm, zeros_hbm, out_hbm,
                  src_v, dst_v, w_v, r0, r1, r2, r3,
                  g0, g1, g2, g3, s0, s1, s2, s3, acc):
        c = lax.axis_index("c")
        s = lax.axis_index("s")
        wid = c * NS + s
        pltpu.sync_copy(src_hbm.at[wid], src_v)
        pltpu.sync_copy(dst_hbm.at[wid], dst_v)
        pltpu.sync_copy(w_hbm.at[wid], w_v)
        pltpu.sync_copy(zeros_hbm.at[pl.ds(s * RPS, RPS)],
                        acc.at[pl.ds(s * RPS, RPS)])
        plsc.subcore_barrier()

        zl = jnp.zeros((16,), jnp.int32)
        rows = (r0, r1, r2, r3)
        gsem = (g0, g1, g2, g3)
        ssem = (s0, s1, s2, s3)

        def gather(j, b):
            pltpu.async_copy(y_hbm.at[src_v.at[j]], rows[b], gsem[b])

        def gwait(j, b):
            pltpu.make_async_copy(y_hbm.at[src_v.at[j]],
                                  rows[b], gsem[b]).wait()

        def scatter(j, b):
            pltpu.async_copy(rows[b], acc.at[dst_v.at[j]], ssem[b], add=True)

        def swait(j, b):
            pltpu.make_async_copy(rows[b], acc.at[dst_v.at[j]],
                                  ssem[b]).wait()

        def scale(j, b):
            jvec = zl + j

            def body(g, cc):
                e0 = g * 8
                sws = [plsc.load_gather(w_v, [jvec, zl + (e0 + i)])
                       for i in range(8)]
                for i in range(8):
                    for f0 in range(0, F, 16):
                        rows[b][e0 + i, pl.ds(f0, 16)] = (
                            rows[b][e0 + i, pl.ds(f0, 16)] * sws[i])
                return cc

            lax.fori_loop(0, C // 8, body, 0)

        gather(0, 0)
        gather(1, 1)
        for j in range(4):
            gwait(j, j)
            scale(j, j)
            scatter(j, j)
            if j < 2:
                gather(j + 2, j + 2)
            else:
                swait(j - 2, j - 2)
                gather(j + 2, j - 2)

        def round_(r, carry):
            for b in range(4):
                j = r * 4 + b
                gwait(j, b)
                scale(j, b)
                scatter(j, b)

                bn = (b + 2) % 4

                @pl.when(j + 2 < chunks)
                def _():
                    swait(j - 2, bn)
                    gather(j + 2, bn)
            return carry

        lax.fori_loop(1, chunks // 4, round_, 0)
        for b in range(4):
            swait(chunks - 4 + b, b)
        plsc.subcore_barrier()
        pltpu.sync_copy(acc.at[pl.ds(s * RPS, RPS)],
                        out_hbm.at[c, pl.ds(s * RPS, RPS)])

    return mp_kernel


_HI = lax.Precision.HIGHEST


def _dot(a, b):
    return lax.dot_general(a, b, (((1,), (0,)), ((), ())),
                           preferred_element_type=jnp.float32)


def _bn_relu(h):
    m = jnp.mean(h, axis=0, keepdims=True)
    cd = h - m
    v = jnp.mean(cd * cd, axis=0, keepdims=True)
    return jnp.maximum(cd / jnp.sqrt(v + 1e-5), 0.0)


def _tca_body(x_ref, w1_ref, xw_ref):
    xw_ref[...] = _dot(x_ref[...], w1_ref[...])


def _tc2_body(sp_ref, xw1_ref, dinv_ref, b1_ref, w2_ref, xw2_ref, y2_ref):
    dinv = dinv_ref[...]
    out = dinv * (sp_ref[0] + sp_ref[1]) + dinv * dinv * xw1_ref[...] \
        + b1_ref[...]
    h = _bn_relu(out)
    xw2 = _dot(h, w2_ref[...])
    xw2_ref[...] = xw2
    y2_ref[...] = xw2 * dinv


def _tc3_body(sp_ref, xw2_ref, dinv_ref, b2_ref, batch_ref,
              fw1a_ref, fw1b_ref, fb1_ref, fw2_ref, fb2_ref, z_ref):
    dinv = dinv_ref[...]
    out = dinv * (sp_ref[0] + sp_ref[1]) + dinv * dinv * xw2_ref[...] \
        + b2_ref[...]
    h = _bn_relu(out)
    b = batch_ref[...]
    iot = lax.broadcasted_iota(jnp.int32, (N, G), 1)
    oh = (b == iot).astype(jnp.float32)
    sums = lax.dot_general(oh, h, (((0,), (0,)), ((), ())),
                           precision=_HI, preferred_element_type=jnp.float32)
    ones = jnp.ones((N, 1), jnp.float32)
    cnt = lax.dot_general(oh, ones, (((0,), (0,)), ((), ())),
                          precision=_HI, preferred_element_type=jnp.float32)
    mean = sums / jnp.maximum(cnt, 1.0)
    neg = jnp.float32(float("-inf"))
    mxs = [jnp.max(jnp.where(b == g, h, neg), axis=0, keepdims=True)
           for g in range(G)]
    mx = jnp.concatenate(mxs, axis=0)
    z1 = _dot(mean, fw1a_ref[...]) + _dot(mx, fw1b_ref[...]) + fb1_ref[...]
    m2 = jnp.mean(z1, axis=0, keepdims=True)
    c2 = z1 - m2
    v2 = jnp.mean(c2 * c2, axis=0, keepdims=True)
    z2 = jnp.maximum(c2 / jnp.sqrt(v2 + 1e-5), 0.0)
    z_ref[...] = _dot(z2, fw2_ref[...]) + fb2_ref[...]


def _sd(shape):
    return jax.ShapeDtypeStruct(shape, jnp.float32)


def kernel(x, edge_index, edge_weight, batch, W1, b1, W2, b2,
           fW1, fb1, fW2, fb2):
    E = edge_weight.shape[0]
    H = W1.shape[1]
    chunks = -(-E // (NW * C))
    chunks += chunks % 2
    epad = NW * chunks * C
    pad = epad - E

    src = edge_index[0].astype(jnp.int32)
    dst = edge_index[1].astype(jnp.int32)
    w = edge_weight.astype(jnp.float32)
    ar = jnp.arange(pad, dtype=jnp.int32)
    src_p = jnp.concatenate([src, ar % N]).reshape(NW, chunks, C)
    dst_p = jnp.concatenate([dst, N + ar % (NP - N)]).reshape(NW, chunks, C)
    w_p = jnp.concatenate([w, jnp.zeros((pad,), jnp.float32)]
                          ).reshape(NW, chunks, C)

    zeros1 = jnp.zeros((NP,), jnp.float32)
    zeros_h = jnp.zeros((NP, H), jnp.float32)
    zeros_2h = jnp.zeros((NP, 2 * H), jnp.float32)

    xw1 = pl.pallas_call(_tca_body, out_shape=_sd((N, H)))(x, W1)
    s1, dinv_np = _make_fused1_kernel(chunks, H)(
        xw1, src_p, dst_p, w_p, zeros1, zeros_h)
    dinv = dinv_np[:N].reshape(N, 1)
    xw2, y2 = pl.pallas_call(
        _tc2_body,
        out_shape=[_sd((N, 2 * H)), _sd((N, 2 * H))],
    )(s1[:, :N, :], xw1, dinv, b1.reshape(1, H), W2)

    s2 = _make_mp_kernel(chunks, 2 * H)(y2, src_p, dst_p, w_p, zeros_2h)
    z = pl.pallas_call(
        _tc3_body,
        out_shape=_sd((G, 2)),
    )(s2[:, :N, :], xw2, dinv, b2.reshape(1, 2 * H),
      batch.reshape(N, 1).astype(jnp.int32),
      fW1[:2 * H], fW1[2 * H:], fb1.reshape(1, H), fW2, fb2.reshape(1, 2))
    return z
